# baseline (speedup 1.0000x reference)
import functools

import jax
import jax.numpy as jnp
from jax import lax
from jax.experimental import pallas as pl
from jax.experimental.pallas import tpu as pltpu
from jax.experimental.pallas import tpu_sc as plsc

VOCAB = 1028
C = 9
H = 2048
NC = 2
NS = 16
L = 16
NW = NC * NS

P = 8
GROUP = C * P
HW = H // 2
Q = 2
W = HW // Q


def _build_sc_kernel(n_pos: int):
    per_w = n_pos // NW
    n_groups = per_w // P

    mesh = plsc.VectorSubcoreMesh(core_axis_name="c", subcore_axis_name="s")

    @functools.partial(
        pl.kernel,
        mesh=mesh,
        out_type=jax.ShapeDtypeStruct((n_pos, H), jnp.float32),
        scratch_types=[
            pltpu.VMEM((n_groups * GROUP,), jnp.int32),
            pltpu.VMEM((GROUP, W), jnp.int32),
            pltpu.VMEM((GROUP, W), jnp.int32),
            pltpu.VMEM((2, P, W), jnp.float32),
            pltpu.VMEM((2, P, W), jnp.float32),
            pltpu.SemaphoreType.DMA,
            pltpu.SemaphoreType.DMA,
            pltpu.SemaphoreType.DMA,
            pltpu.SemaphoreType.DMA,
        ],
    )
    def k(idx_hbm, table_hbm, out_hbm, idx_v, rows0, rows1,
          stage0, stage1, gsem0, gsem1, ssem0, ssem1):
        wid = lax.axis_index("s") * NC + lax.axis_index("c")
        base = wid * per_w
        pltpu.sync_copy(
            idx_hbm.at[pl.ds(wid * n_groups * GROUP, n_groups * GROUP)],
            idx_v)

        def fire_gather(t, rows, sem):
            g = lax.bitwise_and(t, n_groups - 1)
            q = lax.shift_right_logical(t, 4)
            pltpu.async_copy(
                table_hbm.at[idx_v.at[pl.ds(g * GROUP, GROUP)],
                             pl.ds(q * W, W)],
                rows, sem)

        def wait_gather(rows, sem):
            pltpu.make_async_copy(
                table_hbm.at[idx_v.at[pl.ds(0, GROUP)], pl.ds(0, W)],
                rows, sem).wait()

        def fire_stores(t, stage, sem):
            g = lax.bitwise_and(t, n_groups - 1)
            q = lax.shift_right_logical(t, 4)
            rsl = pl.ds(base + g * P, P)
            pltpu.async_copy(
                stage.at[0], out_hbm.at[rsl, pl.ds(q * W, W)], sem)
            pltpu.async_copy(
                stage.at[1], out_hbm.at[rsl, pl.ds(HW + q * W, W)], sem)

        def wait_store(stage, sem):
            dummy = out_hbm.at[pl.ds(0, P), pl.ds(0, W)]
            pltpu.make_async_copy(stage.at[0], dummy, sem).wait()
            pltpu.make_async_copy(stage.at[1], dummy, sem).wait()

        def tree(vs):
            while len(vs) > 1:
                nxt = [vs[i] + vs[i + 1] for i in range(0, len(vs) - 1, 2)]
                if len(vs) % 2:
                    nxt.append(vs[-1])
                vs = nxt
            return vs[0]

        himask = jnp.int32(-65536)

        def compute(rows, stage):
            for r in range(P):
                @pl.loop(0, W, step=2 * L)
                def _(j):
                    for jj in (0, L):
                        sl = pl.ds(j + jj, L)
                        los, his = [], []
                        for c in range(C):
                            v = rows[r * C + c, sl]
                            los.append(lax.bitcast_convert_type(
                                v << 16, jnp.float32))
                            his.append(lax.bitcast_convert_type(
                                v & himask, jnp.float32))
                        stage[0, r, sl] = tree(los)
                        stage[1, r, sl] = tree(his)

        n_items = Q * n_groups

        fire_gather(0, rows0, gsem0)

        @pl.loop(0, n_items // 2)
        def _(k2):
            t0 = 2 * k2
            wait_gather(rows0, gsem0)
            fire_gather(t0 + 1, rows1, gsem1)

            @pl.when(k2 > 0)
            def _():
                wait_store(stage0, ssem0)
            compute(rows0, stage0)
            fire_stores(t0, stage0, ssem0)

            @pl.when(k2 < n_items // 2 - 1)
            def _():
                fire_gather(t0 + 2, rows0, gsem0)
            wait_gather(rows1, gsem1)

            @pl.when(k2 > 0)
            def _():
                wait_store(stage1, ssem1)
            compute(rows1, stage1)
            fire_stores(t0 + 1, stage1, ssem1)

        wait_store(stage0, ssem0)
        wait_store(stage1, ssem1)

    return k


def kernel(audio_codes, embed_table):
    b, s, _ = audio_codes.shape
    n_pos = b * s
    offs = jnp.arange(C, dtype=jnp.int32) * VOCAB
    tok = audio_codes.astype(jnp.int32).reshape(n_pos, C) + offs
    idx = tok.reshape(-1)
    u = lax.bitcast_convert_type(embed_table, jnp.uint32)
    r16 = (u + jnp.uint32(0x7FFF) + ((u >> 16) & jnp.uint32(1))) >> 16
    packed = (r16[:, HW:] << 16) | r16[:, :HW]
    table_i32 = lax.bitcast_convert_type(packed, jnp.int32)
    out = _build_sc_kernel(n_pos)(idx, table_i32)
    return out.reshape(b, s, H)
